# Initial kernel scaffold; baseline (speedup 1.0000x reference)
import jax
import jax.numpy as jnp
from jax.experimental import pallas as pl


def kernel(x, reasoning_embeddings, Gw, Gb, Uw, Ub, Vw, Vb):
    raise NotImplementedError("write your pallas kernel here")


# speedup vs baseline: 2.4896x; 2.4896x over previous
import functools

import jax
import jax.numpy as jnp
from jax.experimental import pallas as pl
from jax.experimental.pallas import tpu as pltpu

B = 4096
D = 384
H = 64
R = 1024
NR = 8
K = 2
AUX = 0.05
TB = 256
NCHUNK = 8
CW = R // NCHUNK

_PREC = jax.lax.Precision.HIGHEST


def _dot(a, b, dims):
    return jax.lax.dot_general(a, b, (dims, ((), ())),
                               preferred_element_type=jnp.float32,
                               precision=_PREC)


def _group_indicator(n, g):
    row = jax.lax.broadcasted_iota(jnp.int32, (n, n // g), 0) // g
    col = jax.lax.broadcasted_iota(jnp.int32, (n, n // g), 1)
    return (row == col).astype(jnp.float32)


def _prep_kernel(x_ref, emb_ref, gw_ref, gb_ref, vw_ref, vb_ref,
                 vet_ref, idx1_ref, idx2_ref, g0_ref, g1_ref, aux_ref):
    logits = _dot(x_ref[...], gw_ref[...], ((1,), (1,))) + gb_ref[...]
    iota8 = jax.lax.broadcasted_iota(jnp.int32, (B, NR), 1)
    v1 = jnp.max(logits, axis=1, keepdims=True)
    i1 = jnp.min(jnp.where(logits == v1, iota8, NR), axis=1, keepdims=True)
    masked = jnp.where(iota8 == i1, -jnp.inf, logits)
    v2 = jnp.max(masked, axis=1, keepdims=True)
    i2 = jnp.min(jnp.where(masked == v2, iota8, NR), axis=1, keepdims=True)
    e = jnp.exp(v2 - v1)
    g0 = 1.0 / (1.0 + e)
    g1 = e / (1.0 + e)
    m = v1
    pe = jnp.exp(logits - m)
    probs = pe / jnp.sum(pe, axis=1, keepdims=True)
    expert_probs = jnp.sum(probs, axis=0, keepdims=True) / B
    tmask = (iota8 == i1).astype(jnp.float32) + (iota8 == i2).astype(jnp.float32)
    expert_mask = jnp.sum(tmask, axis=0, keepdims=True) / B
    aux = NR * jnp.sum(expert_probs * expert_mask, axis=1, keepdims=True) * AUX
    aux_ref[...] = aux
    idx1_ref[...] = i1
    idx2_ref[...] = i2
    g0_ref[...] = g0
    g1_ref[...] = g1
    vet = _dot(vw_ref[...], emb_ref[...], ((1,), (1,))) + vb_ref[...]
    g512 = _group_indicator(NR * H, H)
    n2 = _dot(g512, vet * vet, ((0,), (0,)))
    inv = 1.0 / jnp.maximum(jnp.sqrt(n2), 1e-12)
    scale = _dot(g512, inv, ((1,), (0,)))
    vet_ref[...] = vet * scale


def _main_kernel(x_ref, uw_ref, ub_ref, vet_ref, idx1_ref, idx2_ref,
                 g0_ref, g1_ref, u_ref, sel_ref, logp_ref):
    ux = _dot(x_ref[...], uw_ref[...], ((1,), (1,))) + ub_ref[...]
    g512 = _group_indicator(NR * H, H)
    n2 = _dot(ux * ux, g512, ((1,), (0,)))
    inv = 1.0 / jnp.maximum(jnp.sqrt(n2), 1e-12)
    ux = ux * _dot(inv, g512, ((1,), (1,)))
    grp = jax.lax.broadcasted_iota(jnp.int32, (TB, NR * H), 1) // H
    i1 = idx1_ref[...]
    i2 = idx2_ref[...]
    z0 = jnp.where(grp == i1, ux, 0.0)
    z1 = jnp.where(grp == i2, ux, 0.0)
    s0 = _dot(z0, vet_ref[...], ((1,), (0,)))
    s1 = _dot(z1, vet_ref[...], ((1,), (0,)))
    m0 = jnp.max(s0, axis=1, keepdims=True)
    p0 = jnp.exp(s0 - m0)
    p0 = p0 / jnp.sum(p0, axis=1, keepdims=True)
    m1 = jnp.max(s1, axis=1, keepdims=True)
    p1 = jnp.exp(s1 - m1)
    p1 = p1 / jnp.sum(p1, axis=1, keepdims=True)
    rea = g0_ref[...] * p0 + g1_ref[...] * p1
    u = u_ref[...]
    cind = _group_indicator(R, CW)
    csum = _dot(rea, cind, ((1,), (0,)))
    tri8r = jax.lax.broadcasted_iota(jnp.int32, (NCHUNK, NCHUNK), 0)
    tri8c = jax.lax.broadcasted_iota(jnp.int32, (NCHUNK, NCHUNK), 1)
    tri8 = (tri8r <= tri8c).astype(jnp.float32)
    ccs = _dot(csum, tri8, ((1,), (0,)))
    iota8 = jax.lax.broadcasted_iota(jnp.int32, (TB, NCHUNK), 1)
    crossed = ccs > u
    cstar = jnp.min(jnp.where(crossed, iota8, NCHUNK), axis=1, keepdims=True)
    found = cstar < NCHUNK
    prev = ccs - csum
    prevsel = jnp.sum(jnp.where(iota8 == cstar, prev, 0.0), axis=1,
                      keepdims=True)
    chunk = jnp.zeros((TB, CW), jnp.float32)
    for c in range(NCHUNK):
        chunk = chunk + jnp.where(cstar == c, rea[:, c * CW:(c + 1) * CW], 0.0)
    trir = jax.lax.broadcasted_iota(jnp.int32, (CW, CW), 0)
    tric = jax.lax.broadcasted_iota(jnp.int32, (CW, CW), 1)
    tri128 = (trir <= tric).astype(jnp.float32)
    wcs = _dot(chunk, tri128, ((1,), (0,))) + prevsel
    iota128 = jax.lax.broadcasted_iota(jnp.int32, (TB, CW), 1)
    lmin = jnp.min(jnp.where(wcs > u, iota128, CW), axis=1, keepdims=True)
    lsel = jnp.where(lmin >= CW, CW - 1, lmin)
    selected = jnp.where(found, cstar * CW + lsel, 0)
    pick = jnp.sum(jnp.where(iota128 == lsel, chunk, 0.0), axis=1,
                   keepdims=True)
    pick = jnp.where(found, pick, rea[:, 0:1])
    sel_ref[...] = selected
    logp_ref[...] = jnp.log(pick)


@jax.jit
def kernel(x, reasoning_embeddings, Gw, Gb, Uw, Ub, Vw, Vb):
    vw_flat = Vw.reshape(NR * H, D)
    vb_col = Vb.reshape(NR * H, 1)
    uw_flat = Uw.reshape(NR * H, D)
    ub_row = Ub.reshape(1, NR * H)
    gb_row = Gb.reshape(1, NR)

    vet, idx1, idx2, g0, g1, aux = pl.pallas_call(
        _prep_kernel,
        out_shape=[
            jax.ShapeDtypeStruct((NR * H, R), jnp.float32),
            jax.ShapeDtypeStruct((B, 1), jnp.int32),
            jax.ShapeDtypeStruct((B, 1), jnp.int32),
            jax.ShapeDtypeStruct((B, 1), jnp.float32),
            jax.ShapeDtypeStruct((B, 1), jnp.float32),
            jax.ShapeDtypeStruct((1, 1), jnp.float32),
        ],
    )(x, reasoning_embeddings, Gw, gb_row, vw_flat, vb_col)

    rnd = jax.random.uniform(jax.random.key(42), (B, 1), jnp.float32)

    nblk = B // TB
    blk = lambda i: (i, 0)
    const = lambda i: (0, 0)
    sel, logp = pl.pallas_call(
        _main_kernel,
        grid=(nblk,),
        in_specs=[
            pl.BlockSpec((TB, D), blk),
            pl.BlockSpec((NR * H, D), const),
            pl.BlockSpec((1, NR * H), const),
            pl.BlockSpec((NR * H, R), const),
            pl.BlockSpec((TB, 1), blk),
            pl.BlockSpec((TB, 1), blk),
            pl.BlockSpec((TB, 1), blk),
            pl.BlockSpec((TB, 1), blk),
            pl.BlockSpec((TB, 1), blk),
        ],
        out_specs=[
            pl.BlockSpec((TB, 1), blk),
            pl.BlockSpec((TB, 1), blk),
        ],
        out_shape=[
            jax.ShapeDtypeStruct((B, 1), jnp.int32),
            jax.ShapeDtypeStruct((B, 1), jnp.float32),
        ],
    )(x, uw_flat, ub_row, vet, idx1, idx2, g0, g1, rnd)

    return (sel[:, 0], logp, aux[0, 0])
